# baseline (speedup 1.0000x reference)
import functools

import jax
import jax.numpy as jnp
from jax import lax
from jax.experimental import pallas as pl
from jax.experimental.pallas import tpu as pltpu
from jax.experimental.pallas import tpu_sc as plsc

_NC = 2
_NS = 16
_NW = _NC * _NS
_L = 16


def _replicate(src_ref, src_row, dst_ref, gd, d4):
    vecs = [src_ref[src_row, pl.ds(k * _L, _L)] for k in range(d4 // _L)]

    def body(c, carry):
        for k in range(d4 // _L):
            dst_ref[c, pl.ds(k * _L, _L)] = vecs[k]
        return carry

    lax.fori_loop(0, gd, body, 0)


def _sc_body(gd, ng, d4, row_hbm, col_hbm, io_hbm, pair_hbm, out_hbm,
             col_v, io_s, pair_s, row_rep0, row_rep1, io_rep0, io_rep1,
             pair_rep0, pair_rep1, sem, sem_p):
    r_per_w = gd // _NW
    wid = lax.axis_index("s") * _NC + lax.axis_index("c")
    r0 = wid * r_per_w

    row_reps = [row_rep0, row_rep1]
    io_reps = [io_rep0, io_rep1]
    pair_reps = [pair_rep0, pair_rep1]

    def drain_one(s):
        pltpu.make_async_copy(
            col_v, out_hbm.at[0, 0, :, pl.ds(0, d4)], s).wait()

    pltpu.sync_copy(col_hbm, col_v)
    bulk = 0
    for g in range(ng):
        for rl in range(r_per_w):
            pltpu.async_copy(
                col_v, out_hbm.at[g, r0 + rl, :, pl.ds(d4, d4)], sem)
            bulk += 1

    for rl in range(r_per_w):
        pltpu.sync_copy(row_hbm.at[r0 + rl], row_reps[rl].at[0])
        _replicate(row_reps[rl], 0, row_reps[rl], gd, d4)
        for g in range(ng):
            pltpu.async_copy(
                row_reps[rl], out_hbm.at[g, r0 + rl, :, pl.ds(0, d4)], sem)
            bulk += 1

    pltpu.sync_copy(io_hbm, io_s)
    for par in range(2):
        _replicate(io_s, par, io_reps[par], gd, d4)
        for g in range(par, ng, 2):
            for rl in range(r_per_w):
                pltpu.async_copy(
                    io_reps[par],
                    out_hbm.at[g, r0 + rl, :, pl.ds(2 * d4, d4)], sem)
                bulk += 1

    pltpu.sync_copy(pair_hbm, pair_s)
    gen_fired = [0, 0]
    for gen in range(ng // 2):
        pb = gen % 2
        for _ in range(gen_fired[pb]):
            drain_one(sem_p)
        gen_fired[pb] = 0
        _replicate(pair_s, gen, pair_reps[pb], gd, d4)
        for g in (2 * gen, 2 * gen + 1):
            for rl in range(r_per_w):
                pltpu.async_copy(
                    pair_reps[pb],
                    out_hbm.at[g, r0 + rl, :, pl.ds(3 * d4, d4)], sem_p)
                gen_fired[pb] += 1
    for _ in range(gen_fired[0] + gen_fired[1]):
        drain_one(sem_p)
    for _ in range(bulk):
        drain_one(sem)


def kernel(row_table, col_table, io_table, pair_table, num_grids, grid_dim):
    gd = row_table.shape[0]
    ng = pair_table.shape[0] - 1
    d4 = row_table.shape[-1]
    d = 4 * d4

    mesh = plsc.VectorSubcoreMesh(core_axis_name="c", subcore_axis_name="s")
    tile = pltpu.VMEM((gd, d4), jnp.float32)
    sc_fn = pl.kernel(
        functools.partial(_sc_body, gd, ng, d4),
        mesh=mesh,
        out_type=jax.ShapeDtypeStruct((ng, gd, gd, d), row_table.dtype),
        scratch_types=[
            tile,
            pltpu.VMEM(io_table.shape, jnp.float32),
            pltpu.VMEM(pair_table.shape, jnp.float32),
            tile, tile,
            tile, tile,
            tile, tile,
            pltpu.SemaphoreType.DMA,
            pltpu.SemaphoreType.DMA,
        ],
    )
    return sc_fn(row_table, col_table, io_table, pair_table)


# speedup vs baseline: 1.0661x; 1.0661x over previous
import functools

import jax
import jax.numpy as jnp
from jax import lax
from jax.experimental import pallas as pl
from jax.experimental.pallas import tpu as pltpu
from jax.experimental.pallas import tpu_sc as plsc

_NC = 2
_NS = 16
_NW = _NC * _NS
_L = 16


def _replicate(src_ref, src_row, dst_ref, gd, d4):
    vecs = [src_ref[src_row, pl.ds(k * _L, _L)] for k in range(d4 // _L)]

    def body(c, carry):
        for k in range(d4 // _L):
            dst_ref[c, pl.ds(k * _L, _L)] = vecs[k]
        return carry

    lax.fori_loop(0, gd, body, 0)


def _sc_body(gd, ng, d4, row_hbm, col_hbm, io_hbm, pair_hbm, out_hbm,
             col_v, io_s, pair_s, row_rep0, row_rep1, io_rep0, io_rep1,
             pair_rep0, pair_rep1, sem):
    r_per_w = gd // _NW
    wid = lax.axis_index("s") * _NC + lax.axis_index("c")
    r0 = wid * r_per_w

    pltpu.sync_copy(col_hbm, col_v)
    pltpu.sync_copy(io_hbm, io_s)
    pltpu.sync_copy(pair_hbm, pair_s)

    row_reps = [row_rep0, row_rep1]
    for rl in range(r_per_w):
        pltpu.sync_copy(row_hbm.at[r0 + rl], row_reps[rl].at[0])
        _replicate(row_reps[rl], 0, row_reps[rl], gd, d4)
    _replicate(io_s, 0, io_rep0, gd, d4)
    _replicate(io_s, 1, io_rep1, gd, d4)
    io_reps = [io_rep0, io_rep1]
    pair_reps = [pair_rep0, pair_rep1]

    def drain_one():
        pltpu.make_async_copy(
            col_v, out_hbm.at[0, 0, :, pl.ds(0, d4)], sem).wait()

    gen_fired = [0, 0]
    for g in range(ng):
        if g % 2 == 0:
            pb = (g // 2) % 2
            for _ in range(gen_fired[pb]):
                drain_one()
            gen_fired[pb] = 0
            _replicate(pair_s, g // 2, pair_reps[pb], gd, d4)
        pb = (g // 2) % 2
        for rl in range(r_per_w):
            r = r0 + rl
            pltpu.async_copy(
                row_reps[rl], out_hbm.at[g, r, :, pl.ds(0, d4)], sem)
            pltpu.async_copy(
                col_v, out_hbm.at[g, r, :, pl.ds(d4, d4)], sem)
            pltpu.async_copy(
                io_reps[g % 2], out_hbm.at[g, r, :, pl.ds(2 * d4, d4)], sem)
            pltpu.async_copy(
                pair_reps[pb], out_hbm.at[g, r, :, pl.ds(3 * d4, d4)], sem)
            gen_fired[pb] += 4
    for _ in range(gen_fired[0] + gen_fired[1]):
        drain_one()


def kernel(row_table, col_table, io_table, pair_table, num_grids, grid_dim):
    gd = row_table.shape[0]
    ng = pair_table.shape[0] - 1
    d4 = row_table.shape[-1]
    d = 4 * d4

    mesh = plsc.VectorSubcoreMesh(core_axis_name="c", subcore_axis_name="s")
    tile = pltpu.VMEM((gd, d4), jnp.float32)
    sc_fn = pl.kernel(
        functools.partial(_sc_body, gd, ng, d4),
        mesh=mesh,
        out_type=jax.ShapeDtypeStruct((ng, gd, gd, d), row_table.dtype),
        scratch_types=[
            tile,
            pltpu.VMEM(io_table.shape, jnp.float32),
            pltpu.VMEM(pair_table.shape, jnp.float32),
            tile, tile,
            tile, tile,
            tile, tile,
            pltpu.SemaphoreType.DMA,
        ],
    )
    return sc_fn(row_table, col_table, io_table, pair_table)
